# Initial kernel scaffold; baseline (speedup 1.0000x reference)
import jax
import jax.numpy as jnp
from jax.experimental import pallas as pl


def kernel(features, coords, W, b, gamma, beta):
    raise NotImplementedError("write your pallas kernel here")


# speedup vs baseline: 1.8011x; 1.8011x over previous
import jax
import jax.numpy as jnp
from jax.experimental import pallas as pl
from jax.experimental.pallas import tpu as pltpu

_R = 16
_R3 = _R * _R * _R


def _coords_body(coords_ref, nc_ref, pos_ref):
    c = coords_ref[0]
    mean = jnp.mean(c, axis=1, keepdims=True)
    cc = c - mean
    nrm = jnp.sqrt(jnp.sum(cc * cc, axis=0, keepdims=True))
    denom = jnp.max(nrm) * 2.0
    nc = cc / denom + 0.5
    nc = jnp.clip(nc * float(_R), 0.0, float(_R - 1))
    nc_ref[0] = nc
    v = jnp.round(nc).astype(jnp.int32)
    pos_ref[0] = (v[0:1] + v[1:2] * _R + v[2:3] * (_R * _R))


def _scatter_body(pos_ref, feat_ref, bmax_ref, bmin_ref):
    n = feat_ref.shape[1]
    c = feat_ref.shape[2]
    bmax_ref[0] = jnp.full((_R3, c), -jnp.inf, jnp.float32)
    bmin_ref[0] = jnp.full((_R3, c), jnp.inf, jnp.float32)

    def body(i, carry):
        p = pos_ref[0, 0, i]
        row = feat_ref[0, pl.ds(i, 1), :]
        cur = bmax_ref[0, pl.ds(p, 1), :]
        bmax_ref[0, pl.ds(p, 1), :] = jnp.maximum(cur, row)
        curm = bmin_ref[0, pl.ds(p, 1), :]
        bmin_ref[0, pl.ds(p, 1), :] = jnp.minimum(curm, row)
        return carry

    jax.lax.fori_loop(0, n, body, 0)


def _shift_up(x, k, fill):
    pad = jnp.full((k, x.shape[1]), fill, x.dtype)
    return jnp.concatenate([x[k:, :], pad], axis=0)


def _suffix_mm_body(bmax_ref, bmin_ref, wt_ref, b_ref, out_ref, s_ref, sq_ref):
    bm0 = bmax_ref[0]
    bn0 = bmin_ref[0]
    pres = bm0[:, 0:1] > -jnp.inf
    bm = bm0
    bn = bn0
    k = 1
    while k < _R3:
        bm = jnp.maximum(bm, _shift_up(bm, k, -jnp.inf))
        bn = jnp.minimum(bn, _shift_up(bn, k, jnp.inf))
        k *= 2
    iota = jax.lax.broadcasted_iota(jnp.int32, (_R3, 1), 0)
    vmin = jnp.min(jnp.where(pres, iota, _R3))
    mask = pres & (iota != vmin)
    bm = jnp.where(mask, bm, 0.0)
    bn = jnp.where(mask, bn, 0.0)
    fea = jnp.concatenate([bm, bn], axis=1)
    out = jax.lax.dot_general(fea, wt_ref[...],
                              (((1,), (0,)), ((), ())),
                              preferred_element_type=jnp.float32)
    out = out + b_ref[...]
    out_ref[0] = out
    s_ref[0] = jnp.sum(out, axis=0, keepdims=True)
    sq_ref[0] = jnp.sum(out * out, axis=0, keepdims=True)


def _bn_swish_body(x_ref, scale_ref, shift_ref, y_ref):
    x = x_ref[0]
    y = x * scale_ref[...] + shift_ref[...]
    y_ref[0] = y * jax.nn.sigmoid(y)


def kernel(features, coords, W, b, gamma, beta):
    B, C, N = features.shape
    f32 = jnp.float32

    nc, pos = pl.pallas_call(
        _coords_body,
        grid=(B,),
        in_specs=[pl.BlockSpec((1, 3, N), lambda i: (i, 0, 0))],
        out_specs=[pl.BlockSpec((1, 3, N), lambda i: (i, 0, 0)),
                   pl.BlockSpec((1, 1, N), lambda i: (i, 0, 0))],
        out_shape=[jax.ShapeDtypeStruct((B, 3, N), f32),
                   jax.ShapeDtypeStruct((B, 1, N), jnp.int32)],
    )(coords)

    featT = jnp.swapaxes(features, 1, 2)

    bmax, bmin = pl.pallas_call(
        _scatter_body,
        grid=(B,),
        in_specs=[pl.BlockSpec((1, 1, N), lambda i: (i, 0, 0),
                               memory_space=pltpu.SMEM),
                  pl.BlockSpec((1, N, C), lambda i: (i, 0, 0))],
        out_specs=[pl.BlockSpec((1, _R3, C), lambda i: (i, 0, 0)),
                   pl.BlockSpec((1, _R3, C), lambda i: (i, 0, 0))],
        out_shape=[jax.ShapeDtypeStruct((B, _R3, C), f32),
                   jax.ShapeDtypeStruct((B, _R3, C), f32)],
    )(pos, featT)

    Wt = jnp.swapaxes(W, 0, 1)
    brow = b.reshape(1, C)

    outT, s, sq = pl.pallas_call(
        _suffix_mm_body,
        grid=(B,),
        in_specs=[pl.BlockSpec((1, _R3, C), lambda i: (i, 0, 0)),
                  pl.BlockSpec((1, _R3, C), lambda i: (i, 0, 0)),
                  pl.BlockSpec((2 * C, C), lambda i: (0, 0)),
                  pl.BlockSpec((1, C), lambda i: (0, 0))],
        out_specs=[pl.BlockSpec((1, _R3, C), lambda i: (i, 0, 0)),
                   pl.BlockSpec((1, 1, C), lambda i: (i, 0, 0)),
                   pl.BlockSpec((1, 1, C), lambda i: (i, 0, 0))],
        out_shape=[jax.ShapeDtypeStruct((B, _R3, C), f32),
                   jax.ShapeDtypeStruct((B, 1, C), f32),
                   jax.ShapeDtypeStruct((B, 1, C), f32)],
    )(bmax, bmin, Wt, brow)

    cnt = float(B * _R3)
    mean = jnp.sum(s, axis=(0, 1)) / cnt
    var = jnp.sum(sq, axis=(0, 1)) / cnt - mean * mean
    scale = gamma / jnp.sqrt(var + 1e-5)
    shift = beta - mean * scale

    y = pl.pallas_call(
        _bn_swish_body,
        grid=(B,),
        in_specs=[pl.BlockSpec((1, _R3, C), lambda i: (i, 0, 0)),
                  pl.BlockSpec((1, C), lambda i: (0, 0)),
                  pl.BlockSpec((1, C), lambda i: (0, 0))],
        out_specs=pl.BlockSpec((1, _R3, C), lambda i: (i, 0, 0)),
        out_shape=jax.ShapeDtypeStruct((B, _R3, C), f32),
    )(outT, scale.reshape(1, C), shift.reshape(1, C))

    out = jnp.swapaxes(y, 1, 2).reshape(B, C, _R, _R, _R)
    return (out, nc.reshape(B, 3, N))
